# Initial kernel scaffold; baseline (speedup 1.0000x reference)
import jax
import jax.numpy as jnp
from jax.experimental import pallas as pl


def kernel(x, seg, tok_embed, pos_embed, seg_embed, gamma, beta):
    raise NotImplementedError("write your pallas kernel here")


# speedup vs baseline: 2.1676x; 2.1676x over previous
import functools

import jax
import jax.numpy as jnp
from jax import lax
from jax.experimental import pallas as pl
from jax.experimental.pallas import tpu as pltpu
from jax.experimental.pallas import tpu_sc as plsc

_EPS = 1e-5


def _rsqrt(x):
    i = lax.bitcast_convert_type(x, jnp.int32)
    i = jnp.int32(0x5F3759DF) - (i >> 1)
    y = lax.bitcast_convert_type(i, jnp.float32)
    for _ in range(3):
        y = y * (1.5 - 0.5 * x * y * y)
    return y


def _make_sc_kernel(B, L, V, D, NW):
    seq_per_w = B // NW
    KD = D // 16
    chunks = []
    off = 0
    while off < L:
        n = min(128, L - off)
        chunks.append((off, n))
        off += n

    mesh = plsc.VectorSubcoreMesh(
        core_axis_name="c", subcore_axis_name="s",
        num_cores=2, num_subcores=16)

    @functools.partial(
        pl.kernel,
        out_type=jax.ShapeDtypeStruct((B * L, D), jnp.float32),
        mesh=mesh,
        scratch_types=[
            pltpu.VMEM((L,), jnp.int32),
            pltpu.VMEM((L + 16,), jnp.int32),
            pltpu.VMEM((L, D), jnp.float32),
            pltpu.VMEM((L, D), jnp.float32),
            pltpu.VMEM((2, D), jnp.float32),
            pltpu.VMEM((D,), jnp.float32),
            pltpu.VMEM((D,), jnp.float32),
            pltpu.SemaphoreType.DMA,
        ],
        compiler_params=pltpu.CompilerParams(needs_layout_passes=False),
    )
    def k(x_h, seg_h, tok_h, pos_h, sege_h, g_h, b_h, out_h,
          idx_v, seg_v, rows_v, pos_v, sege_v, g_v, b_v, sem):
        cid = lax.axis_index("c")
        sid = lax.axis_index("s")
        wid = sid * 2 + cid

        pltpu.sync_copy(pos_h.at[pl.ds(0, L)], pos_v)
        pltpu.sync_copy(sege_h, sege_v)
        pltpu.sync_copy(g_h, g_v)
        pltpu.sync_copy(b_h, b_v)

        s0k = [sege_v[0, pl.ds(kk * 16, 16)] for kk in range(KD)]
        dsk = [sege_v[1, pl.ds(kk * 16, 16)] - s0k[kk] for kk in range(KD)]
        gk = [g_v[pl.ds(kk * 16, 16)] for kk in range(KD)]
        bk = [b_v[pl.ds(kk * 16, 16)] for kk in range(KD)]

        def seq_body(j, carry):
            base = (wid * seq_per_w + j) * L
            pltpu.sync_copy(x_h.at[pl.ds(base, L)], idx_v)
            pltpu.sync_copy(seg_h.at[pl.ds(base, L)], seg_v.at[pl.ds(0, L)])
            descs = [
                pltpu.async_copy(
                    tok_h.at[idx_v.at[pl.ds(off, n)]],
                    rows_v.at[pl.ds(off, n)], sem)
                for off, n in chunks
            ]
            for d in descs:
                d.wait()

            def row_body(i, c2):
                sgf = seg_v[pl.ds(i, 16)].astype(jnp.float32)[0]
                v = []
                acc = None
                for kk in range(KD):
                    sl = pl.ds(kk * 16, 16)
                    t = rows_v[i, sl] + pos_v[i, sl] + (s0k[kk] + sgf * dsk[kk])
                    v.append(t)
                    acc = t if acc is None else acc + t
                mean = jnp.sum(acc) * (1.0 / D)
                accq = None
                dv = []
                for kk in range(KD):
                    d2 = v[kk] - mean
                    dv.append(d2)
                    q = d2 * d2
                    accq = q if accq is None else accq + q
                var = jnp.sum(accq) * (1.0 / D)
                r = _rsqrt(var + _EPS)
                for kk in range(KD):
                    sl = pl.ds(kk * 16, 16)
                    rows_v[i, sl] = dv[kk] * (r * gk[kk]) + bk[kk]
                return c2

            lax.fori_loop(0, L, row_body, 0)
            pltpu.sync_copy(rows_v, out_h.at[pl.ds(base, L)])
            return carry

        lax.fori_loop(0, seq_per_w, seq_body, 0)

    return k


def kernel(x, seg, tok_embed, pos_embed, seg_embed, gamma, beta):
    B, L = x.shape
    V, D = tok_embed.shape
    NW = 32
    k = _make_sc_kernel(B, L, V, D, NW)
    out = k(x.reshape(B * L), seg.reshape(B * L), tok_embed, pos_embed,
            seg_embed, gamma, beta)
    return out.reshape(B, L, D)


# speedup vs baseline: 4.1241x; 1.9026x over previous
import functools

import jax
import jax.numpy as jnp
from jax import lax
from jax.experimental import pallas as pl
from jax.experimental.pallas import tpu as pltpu
from jax.experimental.pallas import tpu_sc as plsc

_EPS = 1e-5


def _rsqrt(x):
    i = lax.bitcast_convert_type(x, jnp.int32)
    i = jnp.int32(0x5F3759DF) - (i >> 1)
    y = lax.bitcast_convert_type(i, jnp.float32)
    for _ in range(3):
        y = y * (1.5 - 0.5 * x * y * y)
    return y


def _make_sc_kernel(B, L, V, D, NW):
    seq_per_w = B // NW
    KD = D // 16
    chunks = []
    off = 0
    while off < L:
        n = min(128, L - off)
        chunks.append((off, n))
        off += n

    mesh = plsc.VectorSubcoreMesh(
        core_axis_name="c", subcore_axis_name="s",
        num_cores=2, num_subcores=16)

    @functools.partial(
        pl.kernel,
        out_type=jax.ShapeDtypeStruct((B * L, D), jnp.float32),
        mesh=mesh,
        scratch_types=[
            pltpu.VMEM((L,), jnp.int32),
            pltpu.VMEM((L + 16,), jnp.int32),
            pltpu.VMEM((L, D), jnp.float32),
            pltpu.VMEM((L, D), jnp.float32),
            pltpu.VMEM((2, D), jnp.float32),
            pltpu.VMEM((D,), jnp.float32),
            pltpu.VMEM((D,), jnp.float32),
            pltpu.SemaphoreType.DMA,
        ],
        compiler_params=pltpu.CompilerParams(needs_layout_passes=False),
    )
    def k(x_h, seg_h, tok_h, pos_h, sege_h, g_h, b_h, out_h,
          idx_v, seg_v, rows_v, pos_v, sege_v, g_v, b_v, sem):
        cid = lax.axis_index("c")
        sid = lax.axis_index("s")
        wid = sid * 2 + cid

        pltpu.sync_copy(pos_h.at[pl.ds(0, L)], pos_v)
        pltpu.sync_copy(sege_h, sege_v)
        pltpu.sync_copy(g_h, g_v)
        pltpu.sync_copy(b_h, b_v)

        s0k = [sege_v[0, pl.ds(kk * 16, 16)] for kk in range(KD)]
        dsk = [sege_v[1, pl.ds(kk * 16, 16)] - s0k[kk] for kk in range(KD)]
        gk = [g_v[pl.ds(kk * 16, 16)] for kk in range(KD)]
        bk = [b_v[pl.ds(kk * 16, 16)] for kk in range(KD)]

        def seq_body(j, carry):
            base = (wid * seq_per_w + j) * L
            pltpu.sync_copy(x_h.at[pl.ds(base, L)], idx_v)
            pltpu.sync_copy(seg_h.at[pl.ds(base, L)], seg_v.at[pl.ds(0, L)])
            descs = [
                pltpu.async_copy(
                    tok_h.at[idx_v.at[pl.ds(off, n)]],
                    rows_v.at[pl.ds(off, n)], sem)
                for off, n in chunks
            ]
            for d in descs:
                d.wait()

            @plsc.parallel_loop(0, L, 1, unroll=4)
            def row_body(i):
                sgf = seg_v[pl.ds(i, 16)].astype(jnp.float32)[0]
                v = []
                accs = None
                accq = None
                for kk in range(KD):
                    sl = pl.ds(kk * 16, 16)
                    t = rows_v[i, sl] + pos_v[i, sl] + (s0k[kk] + sgf * dsk[kk])
                    v.append(t)
                    accs = t if accs is None else accs + t
                    q = t * t
                    accq = q if accq is None else accq + q
                mean = jnp.sum(accs) * (1.0 / D)
                msq = jnp.sum(accq) * (1.0 / D)
                var = msq - mean * mean
                r = _rsqrt(var + _EPS)
                for kk in range(KD):
                    t1 = r * gk[kk]
                    t2 = bk[kk] - mean * t1
                    rows_v[i, pl.ds(kk * 16, 16)] = v[kk] * t1 + t2
            pltpu.sync_copy(rows_v, out_h.at[pl.ds(base, L)])
            return carry

        lax.fori_loop(0, seq_per_w, seq_body, 0)

    return k


def kernel(x, seg, tok_embed, pos_embed, seg_embed, gamma, beta):
    B, L = x.shape
    V, D = tok_embed.shape
    NW = 32
    k = _make_sc_kernel(B, L, V, D, NW)
    out = k(x.reshape(B * L), seg.reshape(B * L), tok_embed, pos_embed,
            seg_embed, gamma, beta)
    return out.reshape(B, L, D)


# speedup vs baseline: 4.6412x; 1.1254x over previous
import functools

import jax
import jax.numpy as jnp
from jax import lax
from jax.experimental import pallas as pl
from jax.experimental.pallas import tpu as pltpu
from jax.experimental.pallas import tpu_sc as plsc

_EPS = 1e-5


def _rsqrt(x):
    i = lax.bitcast_convert_type(x, jnp.int32)
    i = jnp.int32(0x5F3759DF) - (i >> 1)
    y = lax.bitcast_convert_type(i, jnp.float32)
    for _ in range(3):
        y = y * (1.5 - 0.5 * x * y * y)
    return y


def _make_sc_kernel(B, L, V, D, NW):
    seq_per_w = B // NW
    KD = D // 16
    chunks = []
    off = 0
    while off < L:
        n = min(128, L - off)
        chunks.append((off, n))
        off += n

    mesh = plsc.VectorSubcoreMesh(
        core_axis_name="c", subcore_axis_name="s",
        num_cores=2, num_subcores=16)

    @functools.partial(
        pl.kernel,
        out_type=jax.ShapeDtypeStruct((B * L, D), jnp.float32),
        mesh=mesh,
        scratch_types=[
            pltpu.VMEM((L,), jnp.int32),
            pltpu.VMEM((L,), jnp.int32),
            pltpu.VMEM((L + 16,), jnp.int32),
            pltpu.VMEM((L + 16,), jnp.int32),
            pltpu.VMEM((L, D), jnp.float32),
            pltpu.VMEM((L, D), jnp.float32),
            pltpu.VMEM((L, D), jnp.float32),
            pltpu.VMEM((2, D), jnp.float32),
            pltpu.VMEM((D,), jnp.float32),
            pltpu.VMEM((D,), jnp.float32),
            pltpu.SemaphoreType.DMA,
            pltpu.SemaphoreType.DMA,
            pltpu.SemaphoreType.DMA,
            pltpu.SemaphoreType.DMA,
        ],
        compiler_params=pltpu.CompilerParams(needs_layout_passes=False),
    )
    def k(x_h, seg_h, tok_h, pos_h, sege_h, g_h, b_h, out_h,
          idx0, idx1, sg0, sg1, rw0, rw1, pos_v, sege_v, g_v, b_v,
          gsem0, gsem1, wsem0, wsem1):
        cid = lax.axis_index("c")
        sid = lax.axis_index("s")
        wid = sid * 2 + cid
        idx = [idx0, idx1]
        sgv = [sg0, sg1]
        rows = [rw0, rw1]
        gsem = [gsem0, gsem1]
        wsem = [wsem0, wsem1]

        pltpu.sync_copy(pos_h.at[pl.ds(0, L)], pos_v)
        pltpu.sync_copy(sege_h, sege_v)
        pltpu.sync_copy(g_h, g_v)
        pltpu.sync_copy(b_h, b_v)

        s0k = [sege_v[0, pl.ds(kk * 16, 16)] for kk in range(KD)]
        dsk = [sege_v[1, pl.ds(kk * 16, 16)] - s0k[kk] for kk in range(KD)]
        gk = [g_v[pl.ds(kk * 16, 16)] for kk in range(KD)]
        bk = [b_v[pl.ds(kk * 16, 16)] for kk in range(KD)]

        def seq_base(j):
            return (wid * seq_per_w + j) * L

        def stage_fire(j, b):
            base = seq_base(j)
            pltpu.sync_copy(x_h.at[pl.ds(base, L)], idx[b])
            pltpu.sync_copy(seg_h.at[pl.ds(base, L)], sgv[b].at[pl.ds(0, L)])
            for off, n in chunks:
                pltpu.async_copy(tok_h.at[idx[b].at[pl.ds(off, n)]],
                                 rows[b].at[pl.ds(off, n)], gsem[b])

        def gather_wait(b):
            pltpu.make_async_copy(tok_h.at[pl.ds(0, L)], rows[b],
                                  gsem[b]).wait()

        def wb_fire(j, b):
            pltpu.async_copy(rows[b], out_h.at[pl.ds(seq_base(j), L)], wsem[b])

        def wb_wait(b):
            pltpu.make_async_copy(rows[b], out_h.at[pl.ds(0, L)],
                                  wsem[b]).wait()

        def compute(b):
            rows_v = rows[b]
            seg_v = sgv[b]

            @plsc.parallel_loop(0, L, 1, unroll=4)
            def row_body(i):
                sgf = seg_v[pl.ds(i, 16)].astype(jnp.float32)[0]
                v = []
                accs = None
                accq = None
                for kk in range(KD):
                    sl = pl.ds(kk * 16, 16)
                    t = rows_v[i, sl] + pos_v[i, sl] + (s0k[kk] + sgf * dsk[kk])
                    v.append(t)
                    accs = t if accs is None else accs + t
                    q = t * t
                    accq = q if accq is None else accq + q
                mean = jnp.sum(accs) * (1.0 / D)
                msq = jnp.sum(accq) * (1.0 / D)
                var = msq - mean * mean
                r = _rsqrt(var + _EPS)
                for kk in range(KD):
                    t1 = r * gk[kk]
                    t2 = bk[kk] - mean * t1
                    rows_v[i, pl.ds(kk * 16, 16)] = v[kk] * t1 + t2

        stage_fire(0, 0)

        def pair_body(t, carry):
            for b in (0, 1):
                j = t * 2 + b

                @pl.when(j > 0)
                def _():
                    wb_wait(1 - b)

                @pl.when(j + 1 < seq_per_w)
                def _():
                    stage_fire(j + 1, 1 - b)

                gather_wait(b)
                compute(b)
                wb_fire(j, b)
            return carry

        lax.fori_loop(0, seq_per_w // 2, pair_body, 0)
        wb_wait(1)

    return k


def kernel(x, seg, tok_embed, pos_embed, seg_embed, gamma, beta):
    B, L = x.shape
    V, D = tok_embed.shape
    NW = 32
    k = _make_sc_kernel(B, L, V, D, NW)
    out = k(x.reshape(B * L), seg.reshape(B * L), tok_embed, pos_embed,
            seg_embed, gamma, beta)
    return out.reshape(B, L, D)
